# Initial kernel scaffold; baseline (speedup 1.0000x reference)
import jax
import jax.numpy as jnp
from jax.experimental import pallas as pl


def kernel(t_int, betas):
    raise NotImplementedError("write your pallas kernel here")


# speedup vs baseline: 4.5616x; 4.5616x over previous
import functools

import jax
import jax.numpy as jnp
from jax import lax
from jax.experimental import pallas as pl
from jax.experimental.pallas import tpu as pltpu
from jax.experimental.pallas import tpu_sc as plsc

_LANES = 16
_TABLE_PAD = 1024


@functools.partial(jax.jit, static_argnums=())
def _sc_gather(t_idx, table):
    batch = t_idx.shape[0]
    info = plsc.get_sparse_core_info()
    num_workers = info.num_cores * info.num_subcores
    per_worker = batch // num_workers

    mesh = plsc.VectorSubcoreMesh(core_axis_name="c", subcore_axis_name="s")

    @functools.partial(
        pl.kernel,
        mesh=mesh,
        out_type=jax.ShapeDtypeStruct((batch,), jnp.float32),
        compiler_params=pltpu.CompilerParams(needs_layout_passes=False),
        scratch_types=[
            pltpu.VMEM((per_worker,), jnp.int32),
            pltpu.VMEM((_TABLE_PAD,), jnp.float32),
            pltpu.VMEM((per_worker,), jnp.float32),
        ],
    )
    def gather_kernel(t_hbm, table_hbm, out_hbm, idx_v, table_v, out_v):
        wid = lax.axis_index("s") * info.num_cores + lax.axis_index("c")
        base = wid * per_worker
        pltpu.sync_copy(t_hbm.at[pl.ds(base, per_worker)], idx_v)
        pltpu.sync_copy(table_hbm, table_v)
        for j in range(per_worker // _LANES):
            idx_vec = idx_v[pl.ds(j * _LANES, _LANES)]
            out_v[pl.ds(j * _LANES, _LANES)] = plsc.load_gather(
                table_v, [idx_vec]
            )
        pltpu.sync_copy(out_v, out_hbm.at[pl.ds(base, per_worker)])

    return gather_kernel(t_idx, table)


def kernel(t_int, betas):
    t_idx = t_int.astype(jnp.int32)
    table = jnp.zeros((_TABLE_PAD,), jnp.float32).at[: betas.shape[0]].set(betas)
    return _sc_gather(t_idx, table)


# speedup vs baseline: 4.6303x; 1.0150x over previous
import functools

import jax
import jax.numpy as jnp
from jax import lax
from jax.experimental import pallas as pl
from jax.experimental.pallas import tpu as pltpu
from jax.experimental.pallas import tpu_sc as plsc

_LANES = 16


@jax.jit
def _sc_gather(t_idx, table):
    batch = t_idx.shape[0]
    table_size = table.shape[0]
    info = plsc.get_sparse_core_info()
    num_workers = info.num_cores * info.num_subcores
    per_worker = batch // num_workers

    mesh = plsc.VectorSubcoreMesh(core_axis_name="c", subcore_axis_name="s")

    @functools.partial(
        pl.kernel,
        mesh=mesh,
        out_type=jax.ShapeDtypeStruct((batch,), jnp.float32),
        compiler_params=pltpu.CompilerParams(needs_layout_passes=False),
        scratch_types=[
            pltpu.VMEM((per_worker,), jnp.int32),
            pltpu.VMEM((table_size,), jnp.float32),
            pltpu.VMEM((per_worker,), jnp.float32),
            pltpu.SemaphoreType.DMA,
        ],
    )
    def gather_kernel(t_hbm, table_hbm, out_hbm, idx_v, table_v, out_v, sem):
        wid = lax.axis_index("s") * info.num_cores + lax.axis_index("c")
        base = wid * per_worker
        cp_idx = pltpu.make_async_copy(
            t_hbm.at[pl.ds(base, per_worker)], idx_v, sem
        )
        cp_tab = pltpu.make_async_copy(table_hbm, table_v, sem)
        cp_idx.start()
        cp_tab.start()
        cp_idx.wait()
        cp_tab.wait()
        for j in range(per_worker // _LANES):
            idx_vec = idx_v[pl.ds(j * _LANES, _LANES)]
            out_v[pl.ds(j * _LANES, _LANES)] = plsc.load_gather(
                table_v, [idx_vec]
            )
        pltpu.sync_copy(out_v, out_hbm.at[pl.ds(base, per_worker)])

    return gather_kernel(t_idx, table)


def kernel(t_int, betas):
    return _sc_gather(t_int.astype(jnp.int32), betas)
